# Initial kernel scaffold; baseline (speedup 1.0000x reference)
import jax
import jax.numpy as jnp
from jax.experimental import pallas as pl


def kernel(output, target):
    raise NotImplementedError("write your pallas kernel here")


# speedup vs baseline: 1.1655x; 1.1655x over previous
import jax
import jax.numpy as jnp
from jax.experimental import pallas as pl
from jax.experimental.pallas import tpu as pltpu

_TOTAL = 4 * 4096 * 2048
_ROWS = 16384
_COLS = 2048
_BLOCK_ROWS = 1024
_GRID = _ROWS // _BLOCK_ROWS


def _mse_body(o_ref, t_ref, out_ref):
    o = o_ref[...]
    t = t_ref[...]
    d = o - t
    sq = jnp.where(jnp.abs(t) > 0.0, d * d, 0.0)
    part = jnp.sum(sq)

    @pl.when(pl.program_id(0) == 0)
    def _():
        out_ref[0, 0] = 0.0

    out_ref[0, 0] += part


def kernel(output, target):
    o2 = output.reshape(_ROWS, _COLS)
    t2 = target.reshape(_ROWS, _COLS)
    total = pl.pallas_call(
        _mse_body,
        grid=(_GRID,),
        in_specs=[
            pl.BlockSpec((_BLOCK_ROWS, _COLS), lambda i: (i, 0)),
            pl.BlockSpec((_BLOCK_ROWS, _COLS), lambda i: (i, 0)),
        ],
        out_specs=pl.BlockSpec(memory_space=pltpu.SMEM),
        out_shape=jax.ShapeDtypeStruct((1, 1), jnp.float32),
    )(o2, t2)
    return total[0, 0] / _TOTAL
